# baseline (speedup 1.0000x reference)
import functools

import jax
import jax.numpy as jnp
from jax import lax
from jax.experimental import pallas as pl
from jax.experimental.pallas import tpu as pltpu
from jax.experimental.pallas import tpu_sc as plsc

N = 10000
E = 320000
XD = 128
H = 128
Z = 64
B = 64
DH = 256
OUT = 231

NC = 2
NS = 16
NW = NC * NS

NP = 10240
EP = 327680
E2P = 331776
EB = EP // NW
EB2 = E2P // NW
STR = NP // NS
RROWS = B + 1
RF = RROWS * NP
RSTR = RF // NS

_SC_MESH = plsc.VectorSubcoreMesh(core_axis_name="c", subcore_axis_name="s")


def _deg_body(dst_hbm, zf_hbm, out_hbm, idx_v, ones_v, bnc_v, deg_sh):
    cid = lax.axis_index("c")
    sid = lax.axis_index("s")
    wid = cid * NS + sid
    for i in range(8):
        ones_v[pl.ds(i * 16, 16)] = jnp.full((16,), 1.0, jnp.float32)
    pltpu.sync_copy(zf_hbm.at[pl.ds(sid * STR, STR)], bnc_v)
    pltpu.sync_copy(bnc_v, deg_sh.at[pl.ds(sid * STR, STR)])
    plsc.subcore_barrier()
    base = wid * EB

    def blk(m, carry):
        off = pl.multiple_of(base + m * 128, 128)
        pltpu.sync_copy(dst_hbm.at[pl.ds(off, 128)], idx_v)
        pltpu.sync_copy(ones_v, deg_sh.at[idx_v], add=True)
        return carry

    lax.fori_loop(0, EB // 128, blk, 0)
    plsc.subcore_barrier()
    pltpu.sync_copy(deg_sh.at[pl.ds(sid * STR, STR)], bnc_v)
    pltpu.sync_copy(bnc_v, out_hbm.at[cid, pl.ds(sid * STR, STR)])


_deg_call = pl.kernel(
    _deg_body,
    out_type=jax.ShapeDtypeStruct((NC, NP), jnp.float32),
    mesh=_SC_MESH,
    scratch_types=[
        pltpu.VMEM((128,), jnp.int32),
        pltpu.VMEM((128,), jnp.float32),
        pltpu.VMEM((STR,), jnp.float32),
        pltpu.VMEM_SHARED((NP,), jnp.float32),
    ],
)


BPT = EB // 128


AGG_U = 8
AGG_B0 = 152
AGG_B1 = 8


def _agg_body(tab_hbm, src2_hbm, dst2_hbm, z2_hbm, out_hbm, sidx, didx, rows, acc_sh,
              isem, gsem, ssem):
    cid = lax.axis_index("c")
    sid = lax.axis_index("s")
    for i in range(5):
        off = sid * STR + i * 128
        pltpu.sync_copy(z2_hbm.at[pl.ds(off, 128)], rows.at[i % 2])
        pltpu.sync_copy(rows.at[i % 2], acc_sh.at[pl.ds(off, 128)])
    plsc.subcore_barrier()

    base = jnp.where(cid == 0, sid * AGG_B0, NS * AGG_B0 + sid * AGG_B1)
    nch = jnp.where(cid == 0, AGG_B0 // AGG_U, AGG_B1 // AGG_U)

    def chunk(k, carry):
        b0 = base + k * AGG_U
        idescs = {}
        sdescs = {}
        for j in range(AGG_U):
            idescs[j] = (
                pltpu.async_copy(src2_hbm.at[b0 + j], sidx.at[j], isem),
                pltpu.async_copy(dst2_hbm.at[b0 + j], didx.at[j], isem),
            )
        for j in range(AGG_U):
            if j - 2 >= 0:
                sdescs.pop(j - 2).wait()
            for d in idescs.pop(j):
                d.wait()
            pltpu.async_copy(tab_hbm.at[sidx.at[j]], rows.at[j % 2], gsem).wait()
            sdescs[j] = pltpu.async_copy(rows.at[j % 2], acc_sh.at[didx.at[j]], ssem, add=True)
        sdescs.pop(AGG_U - 2).wait()
        sdescs.pop(AGG_U - 1).wait()
        return carry

    lax.fori_loop(0, nch, chunk, 0)
    plsc.subcore_barrier()
    for i in range(5):
        off = sid * STR + i * 128
        pltpu.sync_copy(acc_sh.at[pl.ds(off, 128)], rows.at[i % 2])
        pltpu.sync_copy(rows.at[i % 2], out_hbm.at[cid, pl.ds(off, 128)])


_agg_call = pl.kernel(
    _agg_body,
    out_type=jax.ShapeDtypeStruct((NC, NP, 128), jnp.float32),
    mesh=_SC_MESH,
    scratch_types=[
        pltpu.VMEM((AGG_U, 128), jnp.int32),
        pltpu.VMEM((AGG_U, 128), jnp.int32),
        pltpu.VMEM((2, 128, 128), jnp.float32),
        pltpu.VMEM_SHARED((NP, 128), jnp.float32),
        pltpu.SemaphoreType.DMA,
        pltpu.SemaphoreType.DMA,
        pltpu.SemaphoreType.DMA,
    ],
)


def _r_body(src_hbm, dst_hbm, batch_hbm, dinv_hbm, zf_hbm, out_hbm,
            sidx, didx, idxf, val, bt, dv, bnc, r_sh):
    cid = lax.axis_index("c")
    sid = lax.axis_index("s")
    wid = cid * NS + sid
    pltpu.sync_copy(batch_hbm, bt)
    pltpu.sync_copy(dinv_hbm, dv)
    for i in range(5):
        off = sid * RSTR + i * 8320
        pltpu.sync_copy(zf_hbm.at[pl.ds(off, 8320)], bnc)
        pltpu.sync_copy(bnc, r_sh.at[pl.ds(off, 8320)])
    plsc.subcore_barrier()
    base = wid * EB2

    def blk(m, carry):
        off = pl.multiple_of(base + m * 128, 128)
        pltpu.sync_copy(src_hbm.at[pl.ds(off, 128)], sidx)
        pltpu.sync_copy(dst_hbm.at[pl.ds(off, 128)], didx)
        for v in range(8):
            s16 = sidx[pl.ds(v * 16, 16)]
            d16 = didx[pl.ds(v * 16, 16)]
            b16 = plsc.load_gather(bt, [d16])
            w16 = plsc.load_gather(dv, [d16])
            idxf[pl.ds(v * 16, 16)] = b16 * NP + s16
            val[pl.ds(v * 16, 16)] = w16
        pltpu.sync_copy(val, r_sh.at[idxf], add=True)
        return carry

    lax.fori_loop(0, EB2 // 128, blk, 0)
    plsc.subcore_barrier()
    for i in range(5):
        off = sid * RSTR + i * 8320
        pltpu.sync_copy(r_sh.at[pl.ds(off, 8320)], bnc)
        pltpu.sync_copy(bnc, out_hbm.at[cid, pl.ds(off, 8320)])


_r_call = pl.kernel(
    _r_body,
    out_type=jax.ShapeDtypeStruct((NC, RF), jnp.float32),
    mesh=_SC_MESH,
    scratch_types=[
        pltpu.VMEM((128,), jnp.int32),
        pltpu.VMEM((128,), jnp.int32),
        pltpu.VMEM((128,), jnp.int32),
        pltpu.VMEM((128,), jnp.float32),
        pltpu.VMEM((NP,), jnp.int32),
        pltpu.VMEM((NP,), jnp.float32),
        pltpu.VMEM((8320,), jnp.float32),
        pltpu.VMEM_SHARED((RF,), jnp.float32),
    ],
    compiler_params=pltpu.CompilerParams(needs_layout_passes=False),
)


def _selu(v):
    alpha = 1.6732632423543772
    scale = 1.0507009873554805
    return scale * jnp.where(v > 0, v, alpha * (jnp.exp(jnp.minimum(v, 0.0)) - 1.0))


def _silu(v):
    return v * (1.0 / (1.0 + jnp.exp(-v)))


def _logsig(v):
    return jnp.minimum(v, 0.0) - jnp.log1p(jnp.exp(-jnp.abs(v)))


def _scale_body(dinv_ref, x_ref, o_ref):
    o_ref[...] = dinv_ref[...] * x_ref[...]


_scale_call = pl.pallas_call(
    _scale_body,
    grid=(NP // 256,),
    in_specs=[
        pl.BlockSpec((256, 1), lambda i: (i, 0)),
        pl.BlockSpec((256, 128), lambda i: (i, 0)),
    ],
    out_specs=pl.BlockSpec((256, 128), lambda i: (i, 0)),
    out_shape=jax.ShapeDtypeStruct((NP, 128), jnp.float32),
)


def _conv_body(act, rescale, p_ref, t_ref, dinv_ref, w_ref, b_ref, o_ref):
    a = dinv_ref[...] * (p_ref[0] + p_ref[1] + t_ref[...])
    h = act(jnp.dot(a, w_ref[...], preferred_element_type=jnp.float32) + b_ref[...])
    if rescale:
        h = dinv_ref[...] * h
    o_ref[...] = h


def _conv_call(act, rescale):
    return pl.pallas_call(
        functools.partial(_conv_body, act, rescale),
        grid=(NP // 256,),
        in_specs=[
            pl.BlockSpec((2, 256, 128), lambda i: (0, i, 0)),
            pl.BlockSpec((256, 128), lambda i: (i, 0)),
            pl.BlockSpec((256, 1), lambda i: (i, 0)),
            pl.BlockSpec((128, 128), lambda i: (0, 0)),
            pl.BlockSpec((1, 128), lambda i: (0, 0)),
        ],
        out_specs=pl.BlockSpec((256, 128), lambda i: (i, 0)),
        out_shape=jax.ShapeDtypeStruct((NP, 128), jnp.float32),
    )


def _head_body(h2_ref, rp_ref, dinv_ref, batch_ref,
               wmu_ref, bmu_ref, wlv_ref, blv_ref,
               wd0_ref, bd0_ref, wd1_ref, bd1_ref, wd2_ref, bd2_ref,
               wmx_ref, bmx_ref, eps_ref,
               muz_ref, lvz_ref, mux_ref):
    rm = (rp_ref[0] + rp_ref[1]) * dinv_ref[...]
    bids = lax.broadcasted_iota(jnp.int32, (B, NP), 0)
    cnt = jnp.sum(jnp.where(batch_ref[...] == bids, 1.0, 0.0), axis=1, keepdims=True)
    cnt = jnp.maximum(cnt, 1.0)
    ps = jnp.dot(rm, h2_ref[...], preferred_element_type=jnp.float32) / cnt
    mu = jnp.dot(ps, wmu_ref[...], preferred_element_type=jnp.float32) + bmu_ref[...]
    lv = jnp.dot(ps, wlv_ref[...], preferred_element_type=jnp.float32) + blv_ref[...]
    z = mu + jnp.exp(0.5 * lv) * eps_ref[...]
    hd = jnp.tanh(jnp.dot(z, wd0_ref[...], preferred_element_type=jnp.float32) + bd0_ref[...])
    hd = jnp.tanh(jnp.dot(hd, wd1_ref[...], preferred_element_type=jnp.float32) + bd1_ref[...])
    hd = jnp.tanh(jnp.dot(hd, wd2_ref[...], preferred_element_type=jnp.float32) + bd2_ref[...])
    muz_ref[...] = mu
    lvz_ref[...] = lv
    mux_ref[...] = jnp.dot(hd, wmx_ref[...], preferred_element_type=jnp.float32) + bmx_ref[...]


_head_call = pl.pallas_call(
    _head_body,
    out_shape=(
        jax.ShapeDtypeStruct((B, Z), jnp.float32),
        jax.ShapeDtypeStruct((B, Z), jnp.float32),
        jax.ShapeDtypeStruct((B, OUT), jnp.float32),
    ),
)


def kernel(x, edge_index, batch, Wg0, bg0, Wg1, bg1, Wg2, bg2, Wmu, bmu, Wlv, blv,
           Wd0, bd0, Wd1, bd1, Wd2, bd2, Wmx, bmx, logvar_param):
    src = edge_index[0].astype(jnp.int32)
    dst = edge_index[1].astype(jnp.int32)
    batch32 = batch.astype(jnp.int32)

    pad_e = jnp.full((EP - E,), N, jnp.int32)
    src_p = jnp.concatenate([src, pad_e])
    dst_p = jnp.concatenate([dst, pad_e])
    loop = jnp.arange(NP, dtype=jnp.int32)
    pad_e2 = jnp.full((E2P - E - NP,), N, jnp.int32)
    src2 = jnp.concatenate([src, loop, pad_e2])
    dst2 = jnp.concatenate([dst, loop, pad_e2])
    batch_p = jnp.concatenate([batch32, jnp.full((NP - N,), B, jnp.int32)])

    zf = jnp.zeros((RF,), jnp.float32)
    z2 = jnp.zeros((NP, 128), jnp.float32)
    xp = jnp.zeros((NP, 128), jnp.float32).at[:N, :XD].set(x)

    deg_pair = _deg_call(dst_p, zf[:NP])
    deg = deg_pair[0] + deg_pair[1] + 1.0
    dinv = jnp.where(loop < N, lax.rsqrt(jnp.maximum(deg, 1.0)), 0.0)
    dinv_col = dinv[:, None]

    src2d = src_p.reshape(EP // 128, 128)
    dst2d = dst_p.reshape(EP // 128, 128)
    t0 = _scale_call(dinv_col, xp)
    p0 = _agg_call(t0, src2d, dst2d, z2)
    t1 = _conv_call(_selu, True)(p0, t0, dinv_col, Wg0, bg0[None, :])
    p1 = _agg_call(t1, src2d, dst2d, z2)
    t2 = _conv_call(_silu, True)(p1, t1, dinv_col, Wg1, bg1[None, :])
    p2 = _agg_call(t2, src2d, dst2d, z2)
    h2 = _conv_call(_logsig, False)(p2, t2, dinv_col, Wg2, bg2[None, :])

    r_pair = _r_call(src2, dst2, batch_p, dinv, zf)
    rp = r_pair.reshape(NC, RROWS, NP)[:, :B, :]

    eps = jax.random.normal(jax.random.key(42), (B, Z), dtype=jnp.float32)
    mu_z, logvar_z, mu_x = _head_call(
        h2, rp, dinv[None, :], batch_p[None, :],
        Wmu, bmu[None, :], Wlv, blv[None, :],
        Wd0, bd0[None, :], Wd1, bd1[None, :], Wd2, bd2[None, :],
        Wmx, bmx[None, :], eps)

    logvar_x = jnp.tile(logvar_param[None, :], (B, 1))
    return (mu_z, logvar_z, mu_x, logvar_x)


# speedup vs baseline: 1.0258x; 1.0258x over previous
import functools

import jax
import jax.numpy as jnp
from jax import lax
from jax.experimental import pallas as pl
from jax.experimental.pallas import tpu as pltpu
from jax.experimental.pallas import tpu_sc as plsc

N = 10000
E = 320000
XD = 128
H = 128
Z = 64
B = 64
DH = 256
OUT = 231

NC = 2
NS = 16
NW = NC * NS

NP = 10240
EP = 327680
E2P = 331776
EB = EP // NW
EB2 = E2P // NW
STR = NP // NS
RROWS = B + 1
RF = RROWS * NP
RSTR = RF // NS

_SC_MESH = plsc.VectorSubcoreMesh(core_axis_name="c", subcore_axis_name="s")


def _deg_body(dst_hbm, zf_hbm, out_hbm, idx_v, ones_v, bnc_v, deg_sh):
    cid = lax.axis_index("c")
    sid = lax.axis_index("s")
    wid = cid * NS + sid
    for i in range(8):
        ones_v[pl.ds(i * 16, 16)] = jnp.full((16,), 1.0, jnp.float32)
    pltpu.sync_copy(zf_hbm.at[pl.ds(sid * STR, STR)], bnc_v)
    pltpu.sync_copy(bnc_v, deg_sh.at[pl.ds(sid * STR, STR)])
    plsc.subcore_barrier()
    base = wid * EB

    def blk(m, carry):
        off = pl.multiple_of(base + m * 128, 128)
        pltpu.sync_copy(dst_hbm.at[pl.ds(off, 128)], idx_v)
        pltpu.sync_copy(ones_v, deg_sh.at[idx_v], add=True)
        return carry

    lax.fori_loop(0, EB // 128, blk, 0)
    plsc.subcore_barrier()
    pltpu.sync_copy(deg_sh.at[pl.ds(sid * STR, STR)], bnc_v)
    pltpu.sync_copy(bnc_v, out_hbm.at[cid, pl.ds(sid * STR, STR)])


_deg_call = pl.kernel(
    _deg_body,
    out_type=jax.ShapeDtypeStruct((NC, NP), jnp.float32),
    mesh=_SC_MESH,
    scratch_types=[
        pltpu.VMEM((128,), jnp.int32),
        pltpu.VMEM((128,), jnp.float32),
        pltpu.VMEM((STR,), jnp.float32),
        pltpu.VMEM_SHARED((NP,), jnp.float32),
    ],
)


BPT = EB // 128


AGG_U = 8
AGG_B0 = 144
AGG_B1 = 16


def _agg_body(tab_hbm, src2_hbm, dst2_hbm, z2_hbm, out_hbm, sidx, didx, rows, acc_sh,
              isem, gsem, ssem):
    cid = lax.axis_index("c")
    sid = lax.axis_index("s")
    for i in range(5):
        off = sid * STR + i * 128
        pltpu.sync_copy(z2_hbm.at[pl.ds(off, 128)], rows.at[i % 2])
        pltpu.sync_copy(rows.at[i % 2], acc_sh.at[pl.ds(off, 128)])
    plsc.subcore_barrier()

    base = jnp.where(cid == 0, sid * AGG_B0, NS * AGG_B0 + sid * AGG_B1)
    nch = jnp.where(cid == 0, AGG_B0 // AGG_U, AGG_B1 // AGG_U)

    def chunk(k, carry):
        b0 = base + k * AGG_U
        idescs = {}
        sdescs = {}
        for j in range(AGG_U):
            idescs[j] = (
                pltpu.async_copy(src2_hbm.at[b0 + j], sidx.at[j], isem),
                pltpu.async_copy(dst2_hbm.at[b0 + j], didx.at[j], isem),
            )
        for j in range(AGG_U):
            if j - 2 >= 0:
                sdescs.pop(j - 2).wait()
            for d in idescs.pop(j):
                d.wait()
            pltpu.async_copy(tab_hbm.at[sidx.at[j]], rows.at[j % 2], gsem).wait()
            sdescs[j] = pltpu.async_copy(rows.at[j % 2], acc_sh.at[didx.at[j]], ssem, add=True)
        sdescs.pop(AGG_U - 2).wait()
        sdescs.pop(AGG_U - 1).wait()
        return carry

    lax.fori_loop(0, nch, chunk, 0)
    plsc.subcore_barrier()
    for i in range(5):
        off = sid * STR + i * 128
        pltpu.sync_copy(acc_sh.at[pl.ds(off, 128)], rows.at[i % 2])
        pltpu.sync_copy(rows.at[i % 2], out_hbm.at[cid, pl.ds(off, 128)])


_agg_call = pl.kernel(
    _agg_body,
    out_type=jax.ShapeDtypeStruct((NC, NP, 128), jnp.float32),
    mesh=_SC_MESH,
    scratch_types=[
        pltpu.VMEM((AGG_U, 128), jnp.int32),
        pltpu.VMEM((AGG_U, 128), jnp.int32),
        pltpu.VMEM((2, 128, 128), jnp.float32),
        pltpu.VMEM_SHARED((NP, 128), jnp.float32),
        pltpu.SemaphoreType.DMA,
        pltpu.SemaphoreType.DMA,
        pltpu.SemaphoreType.DMA,
    ],
)


def _r_body(src_hbm, dst_hbm, batch_hbm, dinv_hbm, zf_hbm, out_hbm,
            sidx, didx, idxf, val, bt, dv, bnc, r_sh):
    cid = lax.axis_index("c")
    sid = lax.axis_index("s")
    wid = cid * NS + sid
    pltpu.sync_copy(batch_hbm, bt)
    pltpu.sync_copy(dinv_hbm, dv)
    for i in range(5):
        off = sid * RSTR + i * 8320
        pltpu.sync_copy(zf_hbm.at[pl.ds(off, 8320)], bnc)
        pltpu.sync_copy(bnc, r_sh.at[pl.ds(off, 8320)])
    plsc.subcore_barrier()
    base = wid * EB2

    def blk(m, carry):
        off = pl.multiple_of(base + m * 128, 128)
        pltpu.sync_copy(src_hbm.at[pl.ds(off, 128)], sidx)
        pltpu.sync_copy(dst_hbm.at[pl.ds(off, 128)], didx)
        for v in range(8):
            s16 = sidx[pl.ds(v * 16, 16)]
            d16 = didx[pl.ds(v * 16, 16)]
            b16 = plsc.load_gather(bt, [d16])
            w16 = plsc.load_gather(dv, [d16])
            idxf[pl.ds(v * 16, 16)] = b16 * NP + s16
            val[pl.ds(v * 16, 16)] = w16
        pltpu.sync_copy(val, r_sh.at[idxf], add=True)
        return carry

    lax.fori_loop(0, EB2 // 128, blk, 0)
    plsc.subcore_barrier()
    for i in range(5):
        off = sid * RSTR + i * 8320
        pltpu.sync_copy(r_sh.at[pl.ds(off, 8320)], bnc)
        pltpu.sync_copy(bnc, out_hbm.at[cid, pl.ds(off, 8320)])


_r_call = pl.kernel(
    _r_body,
    out_type=jax.ShapeDtypeStruct((NC, RF), jnp.float32),
    mesh=_SC_MESH,
    scratch_types=[
        pltpu.VMEM((128,), jnp.int32),
        pltpu.VMEM((128,), jnp.int32),
        pltpu.VMEM((128,), jnp.int32),
        pltpu.VMEM((128,), jnp.float32),
        pltpu.VMEM((NP,), jnp.int32),
        pltpu.VMEM((NP,), jnp.float32),
        pltpu.VMEM((8320,), jnp.float32),
        pltpu.VMEM_SHARED((RF,), jnp.float32),
    ],
    compiler_params=pltpu.CompilerParams(needs_layout_passes=False),
)


def _selu(v):
    alpha = 1.6732632423543772
    scale = 1.0507009873554805
    return scale * jnp.where(v > 0, v, alpha * (jnp.exp(jnp.minimum(v, 0.0)) - 1.0))


def _silu(v):
    return v * (1.0 / (1.0 + jnp.exp(-v)))


def _logsig(v):
    return jnp.minimum(v, 0.0) - jnp.log1p(jnp.exp(-jnp.abs(v)))


def _scale_body(dinv_ref, x_ref, o_ref):
    o_ref[...] = dinv_ref[...] * x_ref[...]


_scale_call = pl.pallas_call(
    _scale_body,
    grid=(NP // 256,),
    in_specs=[
        pl.BlockSpec((256, 1), lambda i: (i, 0)),
        pl.BlockSpec((256, 128), lambda i: (i, 0)),
    ],
    out_specs=pl.BlockSpec((256, 128), lambda i: (i, 0)),
    out_shape=jax.ShapeDtypeStruct((NP, 128), jnp.float32),
)


def _conv_body(act, rescale, p_ref, t_ref, dinv_ref, w_ref, b_ref, o_ref):
    a = dinv_ref[...] * (p_ref[0] + p_ref[1] + t_ref[...])
    h = act(jnp.dot(a, w_ref[...], preferred_element_type=jnp.float32) + b_ref[...])
    if rescale:
        h = dinv_ref[...] * h
    o_ref[...] = h


def _conv_call(act, rescale):
    return pl.pallas_call(
        functools.partial(_conv_body, act, rescale),
        grid=(NP // 256,),
        in_specs=[
            pl.BlockSpec((2, 256, 128), lambda i: (0, i, 0)),
            pl.BlockSpec((256, 128), lambda i: (i, 0)),
            pl.BlockSpec((256, 1), lambda i: (i, 0)),
            pl.BlockSpec((128, 128), lambda i: (0, 0)),
            pl.BlockSpec((1, 128), lambda i: (0, 0)),
        ],
        out_specs=pl.BlockSpec((256, 128), lambda i: (i, 0)),
        out_shape=jax.ShapeDtypeStruct((NP, 128), jnp.float32),
    )


def _head_body(h2_ref, rp_ref, dinv_ref, batch_ref,
               wmu_ref, bmu_ref, wlv_ref, blv_ref,
               wd0_ref, bd0_ref, wd1_ref, bd1_ref, wd2_ref, bd2_ref,
               wmx_ref, bmx_ref, eps_ref,
               muz_ref, lvz_ref, mux_ref):
    rm = (rp_ref[0] + rp_ref[1]) * dinv_ref[...]
    bids = lax.broadcasted_iota(jnp.int32, (B, NP), 0)
    cnt = jnp.sum(jnp.where(batch_ref[...] == bids, 1.0, 0.0), axis=1, keepdims=True)
    cnt = jnp.maximum(cnt, 1.0)
    ps = jnp.dot(rm, h2_ref[...], preferred_element_type=jnp.float32) / cnt
    mu = jnp.dot(ps, wmu_ref[...], preferred_element_type=jnp.float32) + bmu_ref[...]
    lv = jnp.dot(ps, wlv_ref[...], preferred_element_type=jnp.float32) + blv_ref[...]
    z = mu + jnp.exp(0.5 * lv) * eps_ref[...]
    hd = jnp.tanh(jnp.dot(z, wd0_ref[...], preferred_element_type=jnp.float32) + bd0_ref[...])
    hd = jnp.tanh(jnp.dot(hd, wd1_ref[...], preferred_element_type=jnp.float32) + bd1_ref[...])
    hd = jnp.tanh(jnp.dot(hd, wd2_ref[...], preferred_element_type=jnp.float32) + bd2_ref[...])
    muz_ref[...] = mu
    lvz_ref[...] = lv
    mux_ref[...] = jnp.dot(hd, wmx_ref[...], preferred_element_type=jnp.float32) + bmx_ref[...]


_head_call = pl.pallas_call(
    _head_body,
    out_shape=(
        jax.ShapeDtypeStruct((B, Z), jnp.float32),
        jax.ShapeDtypeStruct((B, Z), jnp.float32),
        jax.ShapeDtypeStruct((B, OUT), jnp.float32),
    ),
)


def kernel(x, edge_index, batch, Wg0, bg0, Wg1, bg1, Wg2, bg2, Wmu, bmu, Wlv, blv,
           Wd0, bd0, Wd1, bd1, Wd2, bd2, Wmx, bmx, logvar_param):
    src = edge_index[0].astype(jnp.int32)
    dst = edge_index[1].astype(jnp.int32)
    batch32 = batch.astype(jnp.int32)

    pad_e = jnp.full((EP - E,), N, jnp.int32)
    src_p = jnp.concatenate([src, pad_e])
    dst_p = jnp.concatenate([dst, pad_e])
    loop = jnp.arange(NP, dtype=jnp.int32)
    pad_e2 = jnp.full((E2P - E - NP,), N, jnp.int32)
    src2 = jnp.concatenate([src, loop, pad_e2])
    dst2 = jnp.concatenate([dst, loop, pad_e2])
    batch_p = jnp.concatenate([batch32, jnp.full((NP - N,), B, jnp.int32)])

    zf = jnp.zeros((RF,), jnp.float32)
    z2 = jnp.zeros((NP, 128), jnp.float32)
    xp = jnp.zeros((NP, 128), jnp.float32).at[:N, :XD].set(x)

    deg_pair = _deg_call(dst_p, zf[:NP])
    deg = deg_pair[0] + deg_pair[1] + 1.0
    dinv = jnp.where(loop < N, lax.rsqrt(jnp.maximum(deg, 1.0)), 0.0)
    dinv_col = dinv[:, None]

    src2d = src_p.reshape(EP // 128, 128)
    dst2d = dst_p.reshape(EP // 128, 128)
    t0 = _scale_call(dinv_col, xp)
    p0 = _agg_call(t0, src2d, dst2d, z2)
    t1 = _conv_call(_selu, True)(p0, t0, dinv_col, Wg0, bg0[None, :])
    p1 = _agg_call(t1, src2d, dst2d, z2)
    t2 = _conv_call(_silu, True)(p1, t1, dinv_col, Wg1, bg1[None, :])
    p2 = _agg_call(t2, src2d, dst2d, z2)
    h2 = _conv_call(_logsig, False)(p2, t2, dinv_col, Wg2, bg2[None, :])

    r_pair = _r_call(src2, dst2, batch_p, dinv, zf)
    rp = r_pair.reshape(NC, RROWS, NP)[:, :B, :]

    eps = jax.random.normal(jax.random.key(42), (B, Z), dtype=jnp.float32)
    mu_z, logvar_z, mu_x = _head_call(
        h2, rp, dinv[None, :], batch_p[None, :],
        Wmu, bmu[None, :], Wlv, blv[None, :],
        Wd0, bd0[None, :], Wd1, bd1[None, :], Wd2, bd2[None, :],
        Wmx, bmx[None, :], eps)

    logvar_x = jnp.tile(logvar_param[None, :], (B, 1))
    return (mu_z, logvar_z, mu_x, logvar_x)


# speedup vs baseline: 1.0491x; 1.0227x over previous
import functools

import jax
import jax.numpy as jnp
from jax import lax
from jax.experimental import pallas as pl
from jax.experimental.pallas import tpu as pltpu
from jax.experimental.pallas import tpu_sc as plsc

N = 10000
E = 320000
XD = 128
H = 128
Z = 64
B = 64
DH = 256
OUT = 231

NC = 2
NS = 16
NW = NC * NS

NP = 10240
EP = 327680
E2P = 331776
EB = EP // NW
EB2 = E2P // NW
STR = NP // NS
RROWS = B + 1
RF = RROWS * NP
RSTR = RF // NS

_SC_MESH = plsc.VectorSubcoreMesh(core_axis_name="c", subcore_axis_name="s")


def _deg_body(dst_hbm, zf_hbm, out_hbm, idx_v, ones_v, bnc_v, deg_sh):
    cid = lax.axis_index("c")
    sid = lax.axis_index("s")
    wid = cid * NS + sid
    for i in range(8):
        ones_v[pl.ds(i * 16, 16)] = jnp.full((16,), 1.0, jnp.float32)
    pltpu.sync_copy(zf_hbm.at[pl.ds(sid * STR, STR)], bnc_v)
    pltpu.sync_copy(bnc_v, deg_sh.at[pl.ds(sid * STR, STR)])
    plsc.subcore_barrier()
    base = wid * EB

    def blk(m, carry):
        off = pl.multiple_of(base + m * 128, 128)
        pltpu.sync_copy(dst_hbm.at[pl.ds(off, 128)], idx_v)
        pltpu.sync_copy(ones_v, deg_sh.at[idx_v], add=True)
        return carry

    lax.fori_loop(0, EB // 128, blk, 0)
    plsc.subcore_barrier()
    pltpu.sync_copy(deg_sh.at[pl.ds(sid * STR, STR)], bnc_v)
    pltpu.sync_copy(bnc_v, out_hbm.at[cid, pl.ds(sid * STR, STR)])


_deg_call = pl.kernel(
    _deg_body,
    out_type=jax.ShapeDtypeStruct((NC, NP), jnp.float32),
    mesh=_SC_MESH,
    scratch_types=[
        pltpu.VMEM((128,), jnp.int32),
        pltpu.VMEM((128,), jnp.float32),
        pltpu.VMEM((STR,), jnp.float32),
        pltpu.VMEM_SHARED((NP,), jnp.float32),
    ],
)


BPT = EB // 128


AGG_U = 8
AGG_B0 = 144
AGG_B1 = 16


def _agg_body(tab_hbm, src2_hbm, dst2_hbm, out_hbm, sidx, didx, rows, acc_sh,
              isem, gsem, ssem):
    cid = lax.axis_index("c")
    sid = lax.axis_index("s")
    z16 = jnp.zeros((16,), jnp.float32)
    for r in range(16):
        for i in range(8):
            rows[0, r, pl.ds(i * 16, 16)] = z16
    for i in range(STR // 16):
        pltpu.sync_copy(rows.at[0, pl.ds(0, 16)], acc_sh.at[pl.ds(sid * STR + i * 16, 16)])
    plsc.subcore_barrier()

    base = jnp.where(cid == 0, sid * AGG_B0, NS * AGG_B0 + sid * AGG_B1)
    nch = jnp.where(cid == 0, AGG_B0 // AGG_U, AGG_B1 // AGG_U)

    def chunk(k, carry):
        b0 = base + k * AGG_U
        idescs = {}
        sdescs = {}
        for j in range(AGG_U):
            idescs[j] = (
                pltpu.async_copy(src2_hbm.at[b0 + j], sidx.at[j], isem),
                pltpu.async_copy(dst2_hbm.at[b0 + j], didx.at[j], isem),
            )
        for j in range(AGG_U):
            if j - 2 >= 0:
                sdescs.pop(j - 2).wait()
            for d in idescs.pop(j):
                d.wait()
            pltpu.async_copy(tab_hbm.at[sidx.at[j]], rows.at[j % 2], gsem).wait()
            sdescs[j] = pltpu.async_copy(rows.at[j % 2], acc_sh.at[didx.at[j]], ssem, add=True)
        sdescs.pop(AGG_U - 2).wait()
        sdescs.pop(AGG_U - 1).wait()
        return carry

    lax.fori_loop(0, nch, chunk, 0)
    plsc.subcore_barrier()
    for i in range(5):
        off = sid * STR + i * 128
        pltpu.sync_copy(acc_sh.at[pl.ds(off, 128)], rows.at[i % 2])
        pltpu.sync_copy(rows.at[i % 2], out_hbm.at[cid, pl.ds(off, 128)])


_agg_call = pl.kernel(
    _agg_body,
    out_type=jax.ShapeDtypeStruct((NC, NP, 128), jnp.float32),
    mesh=_SC_MESH,
    scratch_types=[
        pltpu.VMEM((AGG_U, 128), jnp.int32),
        pltpu.VMEM((AGG_U, 128), jnp.int32),
        pltpu.VMEM((2, 128, 128), jnp.float32),
        pltpu.VMEM_SHARED((NP, 128), jnp.float32),
        pltpu.SemaphoreType.DMA,
        pltpu.SemaphoreType.DMA,
        pltpu.SemaphoreType.DMA,
    ],
)


def _r_body(src_hbm, dst_hbm, batch_hbm, dinv_hbm, out_hbm,
            sidx, didx, idxf, val, bt, dv, bnc, r_sh):
    cid = lax.axis_index("c")
    sid = lax.axis_index("s")
    wid = cid * NS + sid
    pltpu.sync_copy(batch_hbm, bt)
    pltpu.sync_copy(dinv_hbm, dv)
    z16 = jnp.zeros((16,), jnp.float32)
    for i in range(130):
        bnc[pl.ds(i * 16, 16)] = z16
    for i in range(20):
        pltpu.sync_copy(bnc.at[pl.ds(0, 2080)], r_sh.at[pl.ds(sid * RSTR + i * 2080, 2080)])
    plsc.subcore_barrier()
    base = wid * EB2

    def blk(m, carry):
        off = pl.multiple_of(base + m * 128, 128)
        pltpu.sync_copy(src_hbm.at[pl.ds(off, 128)], sidx)
        pltpu.sync_copy(dst_hbm.at[pl.ds(off, 128)], didx)
        for v in range(8):
            s16 = sidx[pl.ds(v * 16, 16)]
            d16 = didx[pl.ds(v * 16, 16)]
            b16 = plsc.load_gather(bt, [d16])
            w16 = plsc.load_gather(dv, [d16])
            idxf[pl.ds(v * 16, 16)] = b16 * NP + s16
            val[pl.ds(v * 16, 16)] = w16
        pltpu.sync_copy(val, r_sh.at[idxf], add=True)
        return carry

    lax.fori_loop(0, EB2 // 128, blk, 0)
    plsc.subcore_barrier()
    for i in range(5):
        off = sid * RSTR + i * 8320
        pltpu.sync_copy(r_sh.at[pl.ds(off, 8320)], bnc)
        pltpu.sync_copy(bnc, out_hbm.at[cid, pl.ds(off, 8320)])


_r_call = pl.kernel(
    _r_body,
    out_type=jax.ShapeDtypeStruct((NC, RF), jnp.float32),
    mesh=_SC_MESH,
    scratch_types=[
        pltpu.VMEM((128,), jnp.int32),
        pltpu.VMEM((128,), jnp.int32),
        pltpu.VMEM((128,), jnp.int32),
        pltpu.VMEM((128,), jnp.float32),
        pltpu.VMEM((NP,), jnp.int32),
        pltpu.VMEM((NP,), jnp.float32),
        pltpu.VMEM((8320,), jnp.float32),
        pltpu.VMEM_SHARED((RF,), jnp.float32),
    ],
    compiler_params=pltpu.CompilerParams(needs_layout_passes=False),
)


def _selu(v):
    alpha = 1.6732632423543772
    scale = 1.0507009873554805
    return scale * jnp.where(v > 0, v, alpha * (jnp.exp(jnp.minimum(v, 0.0)) - 1.0))


def _silu(v):
    return v * (1.0 / (1.0 + jnp.exp(-v)))


def _logsig(v):
    return jnp.minimum(v, 0.0) - jnp.log1p(jnp.exp(-jnp.abs(v)))


def _scale_body(dinv_ref, x_ref, o_ref):
    o_ref[...] = dinv_ref[...] * x_ref[...]


_scale_call = pl.pallas_call(
    _scale_body,
    grid=(NP // 256,),
    in_specs=[
        pl.BlockSpec((256, 1), lambda i: (i, 0)),
        pl.BlockSpec((256, 128), lambda i: (i, 0)),
    ],
    out_specs=pl.BlockSpec((256, 128), lambda i: (i, 0)),
    out_shape=jax.ShapeDtypeStruct((NP, 128), jnp.float32),
)


def _conv_body(act, rescale, p_ref, t_ref, dinv_ref, w_ref, b_ref, o_ref):
    a = dinv_ref[...] * (p_ref[0] + p_ref[1] + t_ref[...])
    h = act(jnp.dot(a, w_ref[...], preferred_element_type=jnp.float32) + b_ref[...])
    if rescale:
        h = dinv_ref[...] * h
    o_ref[...] = h


def _conv_call(act, rescale):
    return pl.pallas_call(
        functools.partial(_conv_body, act, rescale),
        grid=(NP // 256,),
        in_specs=[
            pl.BlockSpec((2, 256, 128), lambda i: (0, i, 0)),
            pl.BlockSpec((256, 128), lambda i: (i, 0)),
            pl.BlockSpec((256, 1), lambda i: (i, 0)),
            pl.BlockSpec((128, 128), lambda i: (0, 0)),
            pl.BlockSpec((1, 128), lambda i: (0, 0)),
        ],
        out_specs=pl.BlockSpec((256, 128), lambda i: (i, 0)),
        out_shape=jax.ShapeDtypeStruct((NP, 128), jnp.float32),
    )


def _head_body(h2_ref, rp_ref, dinv_ref, batch_ref,
               wmu_ref, bmu_ref, wlv_ref, blv_ref,
               wd0_ref, bd0_ref, wd1_ref, bd1_ref, wd2_ref, bd2_ref,
               wmx_ref, bmx_ref, eps_ref,
               muz_ref, lvz_ref, mux_ref):
    rm = (rp_ref[0] + rp_ref[1]) * dinv_ref[...]
    bids = lax.broadcasted_iota(jnp.int32, (B, NP), 0)
    cnt = jnp.sum(jnp.where(batch_ref[...] == bids, 1.0, 0.0), axis=1, keepdims=True)
    cnt = jnp.maximum(cnt, 1.0)
    ps = jnp.dot(rm, h2_ref[...], preferred_element_type=jnp.float32) / cnt
    mu = jnp.dot(ps, wmu_ref[...], preferred_element_type=jnp.float32) + bmu_ref[...]
    lv = jnp.dot(ps, wlv_ref[...], preferred_element_type=jnp.float32) + blv_ref[...]
    z = mu + jnp.exp(0.5 * lv) * eps_ref[...]
    hd = jnp.tanh(jnp.dot(z, wd0_ref[...], preferred_element_type=jnp.float32) + bd0_ref[...])
    hd = jnp.tanh(jnp.dot(hd, wd1_ref[...], preferred_element_type=jnp.float32) + bd1_ref[...])
    hd = jnp.tanh(jnp.dot(hd, wd2_ref[...], preferred_element_type=jnp.float32) + bd2_ref[...])
    muz_ref[...] = mu
    lvz_ref[...] = lv
    mux_ref[...] = jnp.dot(hd, wmx_ref[...], preferred_element_type=jnp.float32) + bmx_ref[...]


_head_call = pl.pallas_call(
    _head_body,
    out_shape=(
        jax.ShapeDtypeStruct((B, Z), jnp.float32),
        jax.ShapeDtypeStruct((B, Z), jnp.float32),
        jax.ShapeDtypeStruct((B, OUT), jnp.float32),
    ),
)


def kernel(x, edge_index, batch, Wg0, bg0, Wg1, bg1, Wg2, bg2, Wmu, bmu, Wlv, blv,
           Wd0, bd0, Wd1, bd1, Wd2, bd2, Wmx, bmx, logvar_param):
    src = edge_index[0].astype(jnp.int32)
    dst = edge_index[1].astype(jnp.int32)
    batch32 = batch.astype(jnp.int32)

    pad_e = jnp.full((EP - E,), N, jnp.int32)
    src_p = jnp.concatenate([src, pad_e])
    dst_p = jnp.concatenate([dst, pad_e])
    loop = jnp.arange(NP, dtype=jnp.int32)
    pad_e2 = jnp.full((E2P - E - NP,), N, jnp.int32)
    src2 = jnp.concatenate([src, loop, pad_e2])
    dst2 = jnp.concatenate([dst, loop, pad_e2])
    batch_p = jnp.concatenate([batch32, jnp.full((NP - N,), B, jnp.int32)])

    zf = jnp.zeros((NP,), jnp.float32)
    xp = jnp.zeros((NP, 128), jnp.float32).at[:N, :XD].set(x)

    deg_pair = _deg_call(dst_p, zf)
    deg = deg_pair[0] + deg_pair[1] + 1.0
    dinv = jnp.where(loop < N, lax.rsqrt(jnp.maximum(deg, 1.0)), 0.0)
    dinv_col = dinv[:, None]

    src2d = src_p.reshape(EP // 128, 128)
    dst2d = dst_p.reshape(EP // 128, 128)
    t0 = _scale_call(dinv_col, xp)
    p0 = _agg_call(t0, src2d, dst2d)
    t1 = _conv_call(_selu, True)(p0, t0, dinv_col, Wg0, bg0[None, :])
    p1 = _agg_call(t1, src2d, dst2d)
    t2 = _conv_call(_silu, True)(p1, t1, dinv_col, Wg1, bg1[None, :])
    p2 = _agg_call(t2, src2d, dst2d)
    h2 = _conv_call(_logsig, False)(p2, t2, dinv_col, Wg2, bg2[None, :])

    r_pair = _r_call(src2, dst2, batch_p, dinv)
    rp = r_pair.reshape(NC, RROWS, NP)[:, :B, :]

    eps = jax.random.normal(jax.random.key(42), (B, Z), dtype=jnp.float32)
    mu_z, logvar_z, mu_x = _head_call(
        h2, rp, dinv[None, :], batch_p[None, :],
        Wmu, bmu[None, :], Wlv, blv[None, :],
        Wd0, bd0[None, :], Wd1, bd1[None, :], Wd2, bd2[None, :],
        Wmx, bmx[None, :], eps)

    logvar_x = jnp.tile(logvar_param[None, :], (B, 1))
    return (mu_z, logvar_z, mu_x, logvar_x)
